# Initial kernel scaffold; baseline (speedup 1.0000x reference)
import jax
import jax.numpy as jnp
from jax.experimental import pallas as pl


def kernel(x, edge_index, W0, W1):
    raise NotImplementedError("write your pallas kernel here")


# speedup vs baseline: 12.7808x; 12.7808x over previous
import jax
import jax.numpy as jnp
from jax import lax
from jax.experimental import pallas as pl
from jax.experimental.pallas import tpu as pltpu
from jax.experimental.pallas import tpu_sc as plsc

_N = 10000
_E = 320000
_D = 128
_C0 = 128
_C1 = 32

_NC = 2
_NS = 16
_NW = _NC * _NS
_B = 128
_NIT = 79
_TE = _B * _NIT
_EP = _TE * _NW
_NP = 10112
_RT = _NP // _NS


def _deg_call(dstp, zeros16, ones16):
  mesh = plsc.VectorSubcoreMesh(core_axis_name="c", subcore_axis_name="s",
                                num_cores=_NC, num_subcores=_NS)

  def body(dst_hbm, zero_hbm, ones_hbm, out_hbm, dstbuf, ones_v, acc, sem):
    cid = lax.axis_index("c")
    sid = lax.axis_index("s")
    wid = sid * _NC + cid
    r0 = sid * _RT
    pltpu.sync_copy(zero_hbm.at[pl.ds(r0, _RT)], acc.at[pl.ds(r0, _RT)])
    pltpu.async_copy(ones_hbm, ones_v, sem).wait()
    plsc.subcore_barrier()

    def step(i, carry):
      base = wid * _TE + i * _B
      pltpu.sync_copy(dst_hbm.at[pl.ds(base, _B)], dstbuf)
      for j in range(_B // 16):
        idx = dstbuf[pl.ds(j * 16, 16)]
        pltpu.sync_copy(ones_v, acc.at[idx], add=True)
      return carry

    lax.fori_loop(0, _NIT, step, 0)
    plsc.subcore_barrier()
    pltpu.sync_copy(acc.at[pl.ds(r0, _RT)], out_hbm.at[cid, pl.ds(r0, _RT)])

  return pl.kernel(
      body,
      out_type=jax.ShapeDtypeStruct((_NC, _NP, _C0), jnp.float32),
      mesh=mesh,
      scratch_types=[
          pltpu.VMEM((_B,), jnp.int32),
          pltpu.VMEM((16, _C0), jnp.float32),
          pltpu.VMEM_SHARED((_NP, _C0), jnp.float32),
          pltpu.SemaphoreType.DMA,
      ],
  )(dstp, zeros16, ones16)


def _prop_call(h, srcp, dstp, zeros, c):
  mesh = plsc.VectorSubcoreMesh(core_axis_name="c", subcore_axis_name="s",
                                num_cores=_NC, num_subcores=_NS)

  def body(h_hbm, src_hbm, dst_hbm, zero_hbm, out_hbm, srcbuf, dstbuf, rows, acc, sem):
    cid = lax.axis_index("c")
    sid = lax.axis_index("s")
    wid = sid * _NC + cid
    r0 = sid * _RT
    pltpu.sync_copy(zero_hbm.at[pl.ds(r0, _RT)], acc.at[pl.ds(r0, _RT)])
    plsc.subcore_barrier()

    def step(i, carry):
      base = wid * _TE + i * _B
      pltpu.sync_copy(src_hbm.at[pl.ds(base, _B)], srcbuf)
      pltpu.sync_copy(dst_hbm.at[pl.ds(base, _B)], dstbuf)
      pltpu.async_copy(h_hbm.at[srcbuf], rows, sem).wait()
      for j in range(_B // 16):
        idx = dstbuf[pl.ds(j * 16, 16)]
        pltpu.sync_copy(rows.at[pl.ds(j * 16, 16)], acc.at[idx], add=True)
      return carry

    lax.fori_loop(0, _NIT, step, 0)
    plsc.subcore_barrier()
    pltpu.sync_copy(acc.at[pl.ds(r0, _RT)], out_hbm.at[cid, pl.ds(r0, _RT)])

  return pl.kernel(
      body,
      out_type=jax.ShapeDtypeStruct((_NC, _NP, c), jnp.float32),
      mesh=mesh,
      scratch_types=[
          pltpu.VMEM((_B,), jnp.int32),
          pltpu.VMEM((_B,), jnp.int32),
          pltpu.VMEM((_B, c), jnp.float32),
          pltpu.VMEM_SHARED((_NP, c), jnp.float32),
          pltpu.SemaphoreType.DMA,
      ],
  )(h, srcp, dstp, zeros)


def _dinv(p_ref):
  deg = p_ref[0, :, 0:1] + p_ref[1, :, 0:1] + 1.0
  return lax.rsqrt(deg)


def _mm0_body(x_ref, w_ref, p_ref, o_ref):
  h = jnp.dot(x_ref[...], w_ref[...], preferred_element_type=jnp.float32)
  o_ref[...] = h * _dinv(p_ref)


def _mid_body(a_ref, h0_ref, w_ref, p_ref, o_ref):
  dinv = _dinv(p_ref)
  z = jnp.maximum((a_ref[0] + a_ref[1] + h0_ref[...]) * dinv, 0.0)
  o_ref[...] = jnp.dot(z, w_ref[...], preferred_element_type=jnp.float32) * dinv


def _fin_body(a_ref, h1_ref, p_ref, o_ref):
  dinv = _dinv(p_ref)
  o_ref[...] = jnp.maximum((a_ref[0, :, :_C1] + a_ref[1, :, :_C1] + h1_ref[:, :_C1]) * dinv, 0.0)


def kernel(x, edge_index, W0, W1):
  src = edge_index[0]
  dst = edge_index[1]
  npad = _NP - _N
  epad = _EP - _E
  padidx = _N + jnp.arange(epad, dtype=jnp.int32) % npad
  srcp = jnp.concatenate([src, padidx])
  dstp = jnp.concatenate([dst, padidx])
  xp = jnp.pad(x, ((0, npad), (0, 0)))

  parts = _deg_call(
      dstp,
      jnp.zeros((_NP, _C0), jnp.float32),
      jnp.ones((16, _C0), jnp.float32),
  )
  h0 = pl.pallas_call(
      _mm0_body,
      out_shape=jax.ShapeDtypeStruct((_NP, _C0), jnp.float32),
  )(xp, W0, parts)
  a = _prop_call(h0, srcp, dstp, jnp.zeros((_NP, _C0), jnp.float32), _C0)
  w1p = jnp.pad(W1, ((0, 0), (0, _C0 - _C1)))
  h1 = pl.pallas_call(
      _mid_body,
      out_shape=jax.ShapeDtypeStruct((_NP, _C0), jnp.float32),
  )(a, h0, w1p, parts)
  a2 = _prop_call(h1, srcp, dstp, jnp.zeros((_NP, _C0), jnp.float32), _C0)
  out = pl.pallas_call(
      _fin_body,
      out_shape=jax.ShapeDtypeStruct((_NP, _C1), jnp.float32),
  )(a2, h1, parts)
  return out[:_N]
